# Initial kernel scaffold; baseline (speedup 1.0000x reference)
import jax
import jax.numpy as jnp
from jax.experimental import pallas as pl


def kernel(x):
    raise NotImplementedError("write your pallas kernel here")


# speedup vs baseline: 46.4226x; 46.4226x over previous
import functools

import jax
import jax.numpy as jnp
from jax.experimental import pallas as pl

_SPARSITY = 0.15
_GAIN = 3.0


def _gated_topk_block(x_ref, o_ref, *, k):
    xb = x_ref[...]
    R, N = xb.shape

    bits = jax.lax.bitcast_convert_type(xb, jnp.uint32)
    sign = bits >= jnp.uint32(0x80000000)
    u = jnp.where(sign, ~bits, bits | jnp.uint32(0x80000000))

    thresh = jnp.zeros((R, 1), dtype=jnp.uint32)
    for bit in range(31, -1, -1):
        cand = thresh | jnp.uint32(1 << bit)
        cnt = jnp.sum((u >= cand).astype(jnp.int32), axis=1, keepdims=True)
        thresh = jnp.where(cnt >= k, cand, thresh)
    mask = u >= thresh

    m1 = jnp.max(xb, axis=1, keepdims=True)
    is_max = xb == m1
    nmax = jnp.sum(is_max.astype(jnp.int32), axis=1, keepdims=True)
    runner = jnp.max(jnp.where(is_max, -jnp.inf, xb), axis=1, keepdims=True)
    m2 = jnp.where(nmax >= 2, m1, runner)
    gain = jax.nn.sigmoid(m1 - m2) * _GAIN + 1.0

    o_ref[...] = jnp.where(mask, xb * gain, 0.0)


@jax.jit
def kernel(x):
    B, N = x.shape
    k = max(int(N * _SPARSITY), 2)
    R = 16
    grid = (B // R,)
    return pl.pallas_call(
        functools.partial(_gated_topk_block, k=k),
        grid=grid,
        in_specs=[pl.BlockSpec((R, N), lambda i: (i, 0))],
        out_specs=pl.BlockSpec((R, N), lambda i: (i, 0)),
        out_shape=jax.ShapeDtypeStruct((B, N), x.dtype),
    )(x)
